# baseline (speedup 1.0000x reference)
import functools

import jax
import jax.numpy as jnp
from jax import lax
from jax.experimental import pallas as pl
from jax.experimental.pallas import tpu as pltpu
from jax.experimental.pallas import tpu_sc as plsc

VOCAB = 100000
D = 64
B = 16384
NC = 2
NS = 16
NW = NC * NS
BPW = B // NW
CHUNK = 128
NCHUNK = BPW // CHUNK
LANES = 16

_mesh = plsc.VectorSubcoreMesh(core_axis_name="c", subcore_axis_name="s")


@functools.partial(
    pl.kernel,
    mesh=_mesh,
    out_type=tuple(jax.ShapeDtypeStruct((B, D), jnp.float32) for _ in range(4)),
    scratch_types=[
        pltpu.VMEM((BPW,), jnp.int32),
        pltpu.VMEM((BPW,), jnp.int32),
        pltpu.VMEM((BPW,), jnp.int32),
        pltpu.VMEM((BPW,), jnp.int32),
        pltpu.VMEM((BPW,), jnp.int32),
        pltpu.VMEM((NCHUNK, CHUNK), jnp.int32),
        pltpu.VMEM((NCHUNK, CHUNK), jnp.int32),
        pltpu.VMEM((NCHUNK, CHUNK), jnp.int32),
        pltpu.VMEM((NCHUNK, CHUNK), jnp.int32),
        pltpu.VMEM((BPW, D), jnp.float32),
        pltpu.VMEM((BPW, D), jnp.float32),
        pltpu.SemaphoreType.DMA,
        pltpu.SemaphoreType.DMA,
        pltpu.SemaphoreType.DMA,
        pltpu.SemaphoreType.DMA,
    ],
    compiler_params=pltpu.CompilerParams(use_tc_tiling_on_sc=False),
)
def _dcl_gather(wi_h, cs_h, co_h, va_h, vb_h, wt_h, ct_h,
                out_wa, out_ca, out_cb, out_wb,
                wi_v, cs_v, co_v, va_v, vb_v,
                iwa, ica, icb, iwb,
                buf0, buf1,
                gsem0, gsem1, wsem0, wsem1):
    wid = lax.axis_index("s") * NC + lax.axis_index("c")
    base = wid * BPW

    pltpu.sync_copy(wi_h.at[pl.ds(base, BPW)], wi_v)
    pltpu.sync_copy(cs_h.at[pl.ds(base, BPW)], cs_v)
    pltpu.sync_copy(co_h.at[pl.ds(base, BPW)], co_v)
    pltpu.sync_copy(va_h.at[pl.ds(base, BPW)], va_v)
    pltpu.sync_copy(vb_h.at[pl.ds(base, BPW)], vb_v)

    for j in range(NCHUNK):
        for k in range(CHUNK // LANES):
            s = pl.ds(j * CHUNK + k * LANES, LANES)
            t = pl.ds(k * LANES, LANES)
            va = va_v[s] * VOCAB
            vb = vb_v[s] * VOCAB
            wi = wi_v[s]
            iwa[j, t] = va + wi
            ica[j, t] = va + cs_v[s]
            icb[j, t] = vb + co_v[s]
            iwb[j, t] = vb + wi

    def fire(table, idxref, buf, sem):
        return [
            pltpu.async_copy(table.at[idxref.at[j]],
                             buf.at[pl.ds(j * CHUNK, CHUNK)], sem)
            for j in range(NCHUNK)
        ]

    g0 = fire(wt_h, iwa, buf0, gsem0)
    g1 = fire(ct_h, ica, buf1, gsem1)
    for c in g0:
        c.wait()
    w0 = pltpu.async_copy(buf0, out_wa.at[pl.ds(base, BPW)], wsem0)
    for c in g1:
        c.wait()
    w1 = pltpu.async_copy(buf1, out_ca.at[pl.ds(base, BPW)], wsem1)
    w0.wait()
    g2 = fire(ct_h, icb, buf0, gsem0)
    w1.wait()
    g3 = fire(wt_h, iwb, buf1, gsem1)
    for c in g2:
        c.wait()
    w2 = pltpu.async_copy(buf0, out_cb.at[pl.ds(base, BPW)], wsem0)
    for c in g3:
        c.wait()
    w3 = pltpu.async_copy(buf1, out_wb.at[pl.ds(base, BPW)], wsem1)
    w2.wait()
    w3.wait()


def kernel(word_idx, ctx_same, ctx_other, variety_a, variety_b,
           word_table, ctx_table):
    i32 = jnp.int32
    return _dcl_gather(
        word_idx.astype(i32), ctx_same.astype(i32), ctx_other.astype(i32),
        variety_a.astype(i32), variety_b.astype(i32),
        word_table, ctx_table)


# speedup vs baseline: 1.0065x; 1.0065x over previous
import functools

import jax
import jax.numpy as jnp
from jax import lax
from jax.experimental import pallas as pl
from jax.experimental.pallas import tpu as pltpu
from jax.experimental.pallas import tpu_sc as plsc

VOCAB = 100000
D = 64
B = 16384
NC = 2
NS = 16
NW = NC * NS
BPW = B // NW
CHUNK = 128
NCHUNK = BPW // CHUNK
LANES = 16

_mesh = plsc.VectorSubcoreMesh(core_axis_name="c", subcore_axis_name="s")


@functools.partial(
    pl.kernel,
    mesh=_mesh,
    out_type=tuple(jax.ShapeDtypeStruct((B, D), jnp.float32) for _ in range(2)),
    scratch_types=[
        pltpu.VMEM((BPW,), jnp.int32),
        pltpu.VMEM((BPW,), jnp.int32),
        pltpu.VMEM((BPW,), jnp.int32),
        pltpu.VMEM((BPW,), jnp.int32),
        pltpu.VMEM((NCHUNK, CHUNK), jnp.int32),
        pltpu.VMEM((NCHUNK, CHUNK), jnp.int32),
        pltpu.VMEM((BPW, D), jnp.float32),
        pltpu.VMEM((BPW, D), jnp.float32),
        pltpu.SemaphoreType.DMA,
        pltpu.SemaphoreType.DMA,
        pltpu.SemaphoreType.DMA,
        pltpu.SemaphoreType.DMA,
    ],
    compiler_params=pltpu.CompilerParams(use_tc_tiling_on_sc=False),
)
def _pair_gather(tok0_h, tok1_h, va_h, vb_h, tab_h,
                 out0, out1,
                 t0_v, t1_v, va_v, vb_v,
                 i0, i1,
                 buf0, buf1,
                 gsem0, gsem1, wsem0, wsem1):
    wid = lax.axis_index("s") * NC + lax.axis_index("c")
    base = wid * BPW

    pltpu.sync_copy(tok0_h.at[pl.ds(base, BPW)], t0_v)
    pltpu.sync_copy(tok1_h.at[pl.ds(base, BPW)], t1_v)
    pltpu.sync_copy(va_h.at[pl.ds(base, BPW)], va_v)
    pltpu.sync_copy(vb_h.at[pl.ds(base, BPW)], vb_v)

    for j in range(NCHUNK):
        for k in range(CHUNK // LANES):
            s = pl.ds(j * CHUNK + k * LANES, LANES)
            t = pl.ds(k * LANES, LANES)
            i0[j, t] = va_v[s] * VOCAB + t0_v[s]
            i1[j, t] = vb_v[s] * VOCAB + t1_v[s]

    def fire(idxref, buf, sem):
        return [
            pltpu.async_copy(tab_h.at[idxref.at[j]],
                             buf.at[pl.ds(j * CHUNK, CHUNK)], sem)
            for j in range(NCHUNK)
        ]

    g0 = fire(i0, buf0, gsem0)
    g1 = fire(i1, buf1, gsem1)
    for c in g0:
        c.wait()
    w0 = pltpu.async_copy(buf0, out0.at[pl.ds(base, BPW)], wsem0)
    for c in g1:
        c.wait()
    w1 = pltpu.async_copy(buf1, out1.at[pl.ds(base, BPW)], wsem1)
    w0.wait()
    w1.wait()


def kernel(word_idx, ctx_same, ctx_other, variety_a, variety_b,
           word_table, ctx_table):
    i32 = jnp.int32
    wi = word_idx.astype(i32)
    va = variety_a.astype(i32)
    vb = variety_b.astype(i32)
    word_emb_a, word_emb_b = _pair_gather(wi, wi, va, vb, word_table)
    ctx_emb_a, ctx_emb_b = _pair_gather(
        ctx_same.astype(i32), ctx_other.astype(i32), va, vb, ctx_table)
    return (word_emb_a, ctx_emb_a, ctx_emb_b, word_emb_b)


# speedup vs baseline: 1.4927x; 1.4831x over previous
import functools

import jax
import jax.numpy as jnp
from jax import lax
from jax.experimental import pallas as pl
from jax.experimental.pallas import tpu as pltpu
from jax.experimental.pallas import tpu_sc as plsc

VOCAB = 100000
D = 64
DP = 128
B = 16384
NT = 2 * B
NC = 2
NS = 16
NW = NC * NS
SPT = NT // NW
SB = 256
NSB = SPT // SB
BPW = B // NW
CHUNK = 128
NCHUNK = BPW // CHUNK
LANES = 16

_mesh = plsc.VectorSubcoreMesh(core_axis_name="c", subcore_axis_name="s")


@functools.partial(
    pl.kernel,
    mesh=_mesh,
    out_type=jax.ShapeDtypeStruct((NT, DP), jnp.float32),
    scratch_types=[
        pltpu.VMEM((SPT,), jnp.int32),
        pltpu.VMEM((64, DP), jnp.float32),
        pltpu.VMEM((SB, DP), jnp.float32),
        pltpu.VMEM((SB, DP), jnp.float32),
        pltpu.SemaphoreType.DMA,
    ],
    compiler_params=pltpu.CompilerParams(use_tc_tiling_on_sc=True,
                                         needs_layout_passes=False),
)
def _extract(t_h, s_h, out_h, s_v, tbuf, rbuf0, rbuf1, wsem):
    wid = lax.axis_index("s") * NC + lax.axis_index("c")
    base = wid * SPT
    pltpu.sync_copy(s_h.at[pl.ds(base, SPT)], s_v)

    def lane_scalar(vec, lane):
        return lax.squeeze(lax.slice(vec, (lane,), (lane + 1,)), (0,))

    cvecs = [q * LANES + lax.iota(jnp.int32, LANES) for q in range(4)]
    rbufs = (rbuf0, rbuf1)
    writes = []
    prev_rg = jnp.int32(-1)
    for sb in range(NSB):
        rbuf = rbufs[sb % 2]
        if len(writes) >= 2:
            writes[sb - 2].wait()

        def group_body(g, prev, rbuf=rbuf, sb=sb):
            sv = s_v[pl.ds(sb * SB + g * LANES, LANES)]
            rgv = lax.shift_right_logical(sv, 7)
            colv = lax.bitwise_and(sv, 127)
            for lane in range(LANES):
                rg = lane_scalar(rgv, lane)

                @pl.when(rg != prev)
                def _():
                    pltpu.sync_copy(t_h.at[:, pl.ds(rg * 128, 128)], tbuf)

                colsplat = jnp.full((LANES,), lane_scalar(colv, lane),
                                    jnp.int32)
                row = g * LANES + lane
                for q in range(4):
                    vals = plsc.load_gather(tbuf, [cvecs[q], colsplat])
                    rbuf[row, pl.ds(q * LANES, LANES)] = vals
                prev = rg
            return prev

        prev_rg = lax.fori_loop(0, SB // LANES, group_body, prev_rg)
        writes.append(
            pltpu.async_copy(rbuf, out_h.at[pl.ds(base + sb * SB, SB)], wsem))
    for w in writes[-2:]:
        w.wait()


@functools.partial(
    pl.kernel,
    mesh=_mesh,
    out_type=tuple(jax.ShapeDtypeStruct((B, DP), jnp.float32) for _ in range(2)),
    scratch_types=[
        pltpu.VMEM((BPW,), jnp.int32),
        pltpu.VMEM((8, CHUNK), jnp.int32),
        pltpu.VMEM((BPW, DP), jnp.float32),
        pltpu.SemaphoreType.DMA,
        pltpu.SemaphoreType.DMA,
    ],
    compiler_params=pltpu.CompilerParams(use_tc_tiling_on_sc=True),
)
def _permute(r_h, ia_h, ib_h, oa, ob, iv, i2d, buf, gsem, wsem):
    wid = lax.axis_index("s") * NC + lax.axis_index("c")
    base = wid * BPW
    for ih, out in ((ia_h, oa), (ib_h, ob)):
        pltpu.sync_copy(ih.at[pl.ds(base, BPW)], iv)
        for k in range(BPW // LANES):
            i2d[k // 8, pl.ds((k % 8) * LANES, LANES)] = iv[pl.ds(k * LANES, LANES)]
        g = [
            pltpu.async_copy(r_h.at[i2d.at[j]],
                             buf.at[pl.ds(j * CHUNK, CHUNK)], gsem)
            for j in range(NCHUNK)
        ]
        for c in g:
            c.wait()
        pltpu.async_copy(buf, out.at[pl.ds(base, BPW)], wsem).wait()


def kernel(word_idx, ctx_same, ctx_other, variety_a, variety_b,
           word_table, ctx_table):
    i32 = jnp.int32
    wi = word_idx.astype(i32)
    va = variety_a.astype(i32)
    vb = variety_b.astype(i32)
    fw = jnp.concatenate([va * VOCAB + wi, vb * VOCAB + wi])
    fc = jnp.concatenate([va * VOCAB + ctx_same.astype(i32),
                          vb * VOCAB + ctx_other.astype(i32)])
    pw = jnp.argsort(fw).astype(i32)
    pc = jnp.argsort(fc).astype(i32)
    sw = jnp.take(fw, pw)
    sc = jnp.take(fc, pc)
    slots = jnp.arange(NT, dtype=i32)
    invw = jnp.zeros((NT,), i32).at[pw].set(slots)
    invc = jnp.zeros((NT,), i32).at[pc].set(slots)

    rows_w = _extract(word_table.T, sw)
    rows_c = _extract(ctx_table.T, sc)
    word_emb_a, word_emb_b = _permute(rows_w, invw[:B], invw[B:])
    ctx_emb_a, ctx_emb_b = _permute(rows_c, invc[:B], invc[B:])
    return (word_emb_a[:, :D], ctx_emb_a[:, :D],
            ctx_emb_b[:, :D], word_emb_b[:, :D])


# speedup vs baseline: 2.1720x; 1.4550x over previous
import functools

import jax
import jax.numpy as jnp
from jax import lax
from jax.experimental import pallas as pl
from jax.experimental.pallas import tpu as pltpu
from jax.experimental.pallas import tpu_sc as plsc

VOCAB = 100000
D = 64
DP = 128
B = 16384
NT = 2 * B
NC = 2
NS = 16
NW = NC * NS
SPT = NT // NW
SB = 256
NSB = SPT // SB
BPW = B // NW
CHUNK = 128
NCHUNK = BPW // CHUNK
LANES = 16
WIN = 512
WSHIFT = 9
VROWS = 8 * VOCAB

_mesh = plsc.VectorSubcoreMesh(core_axis_name="c", subcore_axis_name="s")


@functools.partial(
    pl.kernel,
    mesh=_mesh,
    out_type=jax.ShapeDtypeStruct((NT, DP), jnp.float32),
    scratch_types=[
        pltpu.VMEM((SPT,), jnp.int32),
        pltpu.VMEM((64, WIN), jnp.float32),
        pltpu.VMEM((SB, DP), jnp.float32),
        pltpu.VMEM((SB, DP), jnp.float32),
        pltpu.SemaphoreType.DMA,
    ],
    compiler_params=pltpu.CompilerParams(use_tc_tiling_on_sc=True,
                                         needs_layout_passes=False),
)
def _extract(t_h, s_h, out_h, s_v, tbuf, rbuf0, rbuf1, wsem):
    wid = lax.axis_index("s") * NC + lax.axis_index("c")
    base = wid * SPT
    pltpu.sync_copy(s_h.at[pl.ds(base, SPT)], s_v)

    def lane_scalar(vec, lane):
        return lax.squeeze(lax.slice(vec, (lane,), (lane + 1,)), (0,))

    cvecs = [q * LANES + lax.iota(jnp.int32, LANES) for q in range(4)]
    rbufs = (rbuf0, rbuf1)
    writes = []
    prev_rg = jnp.int32(-1)
    for sb in range(NSB):
        rbuf = rbufs[sb % 2]
        if len(writes) >= 2:
            writes[sb - 2].wait()

        def group_body(g, prev, rbuf=rbuf, sb=sb):
            sv = s_v[pl.ds(sb * SB + g * LANES, LANES)]
            for lane in range(LANES):
                sflat = lane_scalar(sv, lane)
                win = lax.shift_right_logical(sflat, WSHIFT)

                @pl.when(win != prev)
                def _():
                    wstart = lax.min(win * WIN, VROWS - WIN)
                    pltpu.sync_copy(t_h.at[:, pl.ds(wstart, WIN)], tbuf)

                wstart = lax.min(win * WIN, VROWS - WIN)
                colsplat = jnp.full((LANES,), sflat - wstart, jnp.int32)
                row = g * LANES + lane
                for q in range(4):
                    vals = plsc.load_gather(tbuf, [cvecs[q], colsplat])
                    rbuf[row, pl.ds(q * LANES, LANES)] = vals
                prev = win
            return prev

        prev_rg = lax.fori_loop(0, SB // LANES, group_body, prev_rg)
        writes.append(
            pltpu.async_copy(rbuf, out_h.at[pl.ds(base + sb * SB, SB)], wsem))
    for w in writes[-2:]:
        w.wait()


@functools.partial(
    pl.kernel,
    mesh=_mesh,
    out_type=tuple(jax.ShapeDtypeStruct((B, DP), jnp.float32) for _ in range(2)),
    scratch_types=[
        pltpu.VMEM((BPW,), jnp.int32),
        pltpu.VMEM((8, CHUNK), jnp.int32),
        pltpu.VMEM((BPW, DP), jnp.float32),
        pltpu.SemaphoreType.DMA,
        pltpu.SemaphoreType.DMA,
    ],
    compiler_params=pltpu.CompilerParams(use_tc_tiling_on_sc=True),
)
def _permute(r_h, ia_h, ib_h, oa, ob, iv, i2d, buf, gsem, wsem):
    wid = lax.axis_index("s") * NC + lax.axis_index("c")
    base = wid * BPW
    for ih, out in ((ia_h, oa), (ib_h, ob)):
        pltpu.sync_copy(ih.at[pl.ds(base, BPW)], iv)
        for k in range(BPW // LANES):
            i2d[k // 8, pl.ds((k % 8) * LANES, LANES)] = iv[pl.ds(k * LANES, LANES)]
        g = [
            pltpu.async_copy(r_h.at[i2d.at[j]],
                             buf.at[pl.ds(j * CHUNK, CHUNK)], gsem)
            for j in range(NCHUNK)
        ]
        for c in g:
            c.wait()
        pltpu.async_copy(buf, out.at[pl.ds(base, BPW)], wsem).wait()


def kernel(word_idx, ctx_same, ctx_other, variety_a, variety_b,
           word_table, ctx_table):
    i32 = jnp.int32
    wi = word_idx.astype(i32)
    va = variety_a.astype(i32)
    vb = variety_b.astype(i32)
    fw = jnp.concatenate([va * VOCAB + wi, vb * VOCAB + wi])
    fc = jnp.concatenate([va * VOCAB + ctx_same.astype(i32),
                          vb * VOCAB + ctx_other.astype(i32)])
    pw = jnp.argsort(fw).astype(i32)
    pc = jnp.argsort(fc).astype(i32)
    sw = jnp.take(fw, pw)
    sc = jnp.take(fc, pc)
    slots = jnp.arange(NT, dtype=i32)
    invw = jnp.zeros((NT,), i32).at[pw].set(slots)
    invc = jnp.zeros((NT,), i32).at[pc].set(slots)

    rows_w = _extract(word_table.T, sw)
    rows_c = _extract(ctx_table.T, sc)
    word_emb_a, word_emb_b = _permute(rows_w, invw[:B], invw[B:])
    ctx_emb_a, ctx_emb_b = _permute(rows_c, invc[:B], invc[B:])
    return (word_emb_a[:, :D], ctx_emb_a[:, :D],
            ctx_emb_b[:, :D], word_emb_b[:, :D])


# speedup vs baseline: 2.3010x; 1.0594x over previous
import functools

import jax
import jax.numpy as jnp
from jax import lax
from jax.experimental import pallas as pl
from jax.experimental.pallas import tpu as pltpu
from jax.experimental.pallas import tpu_sc as plsc

VOCAB = 100000
D = 64
DP = 128
B = 16384
NT = 2 * B
NC = 2
NS = 16
NW = NC * NS
SPT = NT // NW
SB = 128
NSB = SPT // SB
BPW = B // NW
CHUNK = 128
NCHUNK = BPW // CHUNK
LANES = 16
WIN = 1024
WSHIFT = 10
VROWS = 8 * VOCAB

_mesh = plsc.VectorSubcoreMesh(core_axis_name="c", subcore_axis_name="s")


@functools.partial(
    pl.kernel,
    mesh=_mesh,
    out_type=jax.ShapeDtypeStruct((NT, DP), jnp.float32),
    scratch_types=[
        pltpu.VMEM((SPT,), jnp.int32),
        pltpu.VMEM((64, WIN), jnp.float32),
        pltpu.VMEM((SB, DP), jnp.float32),
        pltpu.VMEM((SB, DP), jnp.float32),
        pltpu.SemaphoreType.DMA,
    ],
    compiler_params=pltpu.CompilerParams(use_tc_tiling_on_sc=True,
                                         needs_layout_passes=False),
)
def _extract(t_h, s_h, out_h, s_v, tbuf, rbuf0, rbuf1, wsem):
    wid = lax.axis_index("s") * NC + lax.axis_index("c")
    base = wid * SPT
    pltpu.sync_copy(s_h.at[pl.ds(base, SPT)], s_v)

    def lane_scalar(vec, lane):
        return lax.squeeze(lax.slice(vec, (lane,), (lane + 1,)), (0,))

    cvecs = [q * LANES + lax.iota(jnp.int32, LANES) for q in range(4)]
    rbufs = (rbuf0, rbuf1)
    writes = []
    prev_rg = jnp.int32(-1)
    for sb in range(NSB):
        rbuf = rbufs[sb % 2]
        if len(writes) >= 2:
            writes[sb - 2].wait()

        def group_body(g, prev, rbuf=rbuf, sb=sb):
            sv = s_v[pl.ds(sb * SB + g * LANES, LANES)]
            for lane in range(LANES):
                sflat = lane_scalar(sv, lane)
                win = lax.shift_right_logical(sflat, WSHIFT)

                @pl.when(win != prev)
                def _():
                    wstart = lax.min(win * WIN, VROWS - WIN)
                    pltpu.sync_copy(t_h.at[:, pl.ds(wstart, WIN)], tbuf)

                wstart = lax.min(win * WIN, VROWS - WIN)
                colsplat = jnp.full((LANES,), sflat - wstart, jnp.int32)
                row = g * LANES + lane
                for q in range(4):
                    vals = plsc.load_gather(tbuf, [cvecs[q], colsplat])
                    rbuf[row, pl.ds(q * LANES, LANES)] = vals
                prev = win
            return prev

        prev_rg = lax.fori_loop(0, SB // LANES, group_body, prev_rg)
        writes.append(
            pltpu.async_copy(rbuf, out_h.at[pl.ds(base + sb * SB, SB)], wsem))
    for w in writes[-2:]:
        w.wait()


@functools.partial(
    pl.kernel,
    mesh=_mesh,
    out_type=tuple(jax.ShapeDtypeStruct((B, DP), jnp.float32) for _ in range(2)),
    scratch_types=[
        pltpu.VMEM((BPW,), jnp.int32),
        pltpu.VMEM((8, CHUNK), jnp.int32),
        pltpu.VMEM((BPW, DP), jnp.float32),
        pltpu.SemaphoreType.DMA,
        pltpu.SemaphoreType.DMA,
    ],
    compiler_params=pltpu.CompilerParams(use_tc_tiling_on_sc=True),
)
def _permute(r_h, ia_h, ib_h, oa, ob, iv, i2d, buf, gsem, wsem):
    wid = lax.axis_index("s") * NC + lax.axis_index("c")
    base = wid * BPW
    for ih, out in ((ia_h, oa), (ib_h, ob)):
        pltpu.sync_copy(ih.at[pl.ds(base, BPW)], iv)
        for k in range(BPW // LANES):
            i2d[k // 8, pl.ds((k % 8) * LANES, LANES)] = iv[pl.ds(k * LANES, LANES)]
        g = [
            pltpu.async_copy(r_h.at[i2d.at[j]],
                             buf.at[pl.ds(j * CHUNK, CHUNK)], gsem)
            for j in range(NCHUNK)
        ]
        for c in g:
            c.wait()
        pltpu.async_copy(buf, out.at[pl.ds(base, BPW)], wsem).wait()


def kernel(word_idx, ctx_same, ctx_other, variety_a, variety_b,
           word_table, ctx_table):
    i32 = jnp.int32
    wi = word_idx.astype(i32)
    va = variety_a.astype(i32)
    vb = variety_b.astype(i32)
    fw = jnp.concatenate([va * VOCAB + wi, vb * VOCAB + wi])
    fc = jnp.concatenate([va * VOCAB + ctx_same.astype(i32),
                          vb * VOCAB + ctx_other.astype(i32)])
    pw = jnp.argsort(fw).astype(i32)
    pc = jnp.argsort(fc).astype(i32)
    sw = jnp.take(fw, pw)
    sc = jnp.take(fc, pc)
    slots = jnp.arange(NT, dtype=i32)
    invw = jnp.zeros((NT,), i32).at[pw].set(slots)
    invc = jnp.zeros((NT,), i32).at[pc].set(slots)

    rows_w = _extract(word_table.T, sw)
    rows_c = _extract(ctx_table.T, sc)
    word_emb_a, word_emb_b = _permute(rows_w, invw[:B], invw[B:])
    ctx_emb_a, ctx_emb_b = _permute(rows_c, invc[:B], invc[B:])
    return (word_emb_a[:, :D], ctx_emb_a[:, :D],
            ctx_emb_b[:, :D], word_emb_b[:, :D])
